# Initial kernel scaffold; baseline (speedup 1.0000x reference)
import jax
import jax.numpy as jnp
from jax.experimental import pallas as pl


def kernel(user_indices, item_indices, user_emb, item_emb, edge_index, edge_weight):
    raise NotImplementedError("write your pallas kernel here")


# speedup vs baseline: 19.6647x; 19.6647x over previous
import functools

import jax
import jax.numpy as jnp
from jax import lax
from jax.experimental import pallas as pl
from jax.experimental.pallas import tpu as pltpu
from jax.experimental.pallas import tpu_sc as plsc

N_USERS = 50000
N_ITEMS = 50000
N_NODES = N_USERS + N_ITEMS
D = 16
E = 3200000
BATCH = 16384

NC = 2
NS = 16
NW = NC * NS
G = 128
CG = 4
C = G * CG
NCHUNKS = E // C
CH_BASE = NCHUNKS // NW
CH_REM = NCHUNKS - CH_BASE * NW
NPIPE = CH_BASE - 1

RPW = N_NODES // NS
ZC = RPW // C
ZT = RPW - ZC * C

_mesh = plsc.VectorSubcoreMesh(
    core_axis_name="c", subcore_axis_name="s", num_cores=NC, num_subcores=NS)
_sc_params = pltpu.CompilerParams(
    needs_layout_passes=False, use_tc_tiling_on_sc=False)


@functools.partial(
    pl.kernel,
    mesh=_mesh,
    out_type=jax.ShapeDtypeStruct((NC, N_NODES, D), jnp.float32),
    scratch_types=[
        pltpu.VMEM_SHARED((N_NODES, D), jnp.float32),
        pltpu.VMEM((2, C), jnp.int32),
        pltpu.VMEM((2, C), jnp.int32),
        pltpu.VMEM((2, C), jnp.float32),
        pltpu.VMEM((2, C, D), jnp.float32),
        pltpu.SemaphoreType.DMA,
        pltpu.SemaphoreType.DMA,
        pltpu.SemaphoreType.DMA,
    ],
    compiler_params=_sc_params,
)
def _layer(x_hbm, ei_hbm, w_hbm, y_hbm, acc, src_v, dst_v, w_v, rows_v,
           sem_e, sem_g0, sem_g1):
    c = lax.axis_index("c")
    s = lax.axis_index("s")
    wid = c * NS + s
    sem_g = (sem_g0, sem_g1)

    @pl.loop(0, C)
    def _(i):
        rows_v[0, i, :] = jnp.zeros((D,), jnp.float32)

    for j in range(ZC):
        pltpu.sync_copy(rows_v.at[0], acc.at[pl.ds(s * RPW + j * C, C)])
    pltpu.sync_copy(rows_v.at[0].at[pl.ds(0, ZT)],
                    acc.at[pl.ds(s * RPW + ZC * C, ZT)])
    plsc.subcore_barrier()

    ch0 = wid * CH_BASE + jnp.minimum(wid, CH_REM)

    def edge_issue(ch, b):
        pltpu.async_copy(ei_hbm.at[0, ch], src_v.at[b], sem_e)
        pltpu.async_copy(ei_hbm.at[1, ch], dst_v.at[b], sem_e)
        pltpu.async_copy(w_hbm.at[pl.ds(ch * C, C)], w_v.at[b], sem_e)

    def edge_wait(b):
        pltpu.make_async_copy(ei_hbm.at[0, 0], src_v.at[b], sem_e).wait()
        pltpu.make_async_copy(ei_hbm.at[1, 0], dst_v.at[b], sem_e).wait()
        pltpu.make_async_copy(w_hbm.at[pl.ds(0, C)], w_v.at[b], sem_e).wait()

    def gather_issue(b):
        pltpu.async_copy(x_hbm.at[src_v.at[b]], rows_v.at[b], sem_g[b])

    def gather_wait(b):
        pltpu.make_async_copy(x_hbm.at[src_v.at[b]], rows_v.at[b],
                              sem_g[b]).wait()

    def mul(b):
        rows2 = rows_v.at[b]

        @pl.loop(0, C // D)
        def _(t):
            w16 = w_v[b, pl.ds(t * D, D)]
            ridx = lax.iota(jnp.int32, D) + t * D
            for d in range(D):
                dsp = jnp.full((D,), d, jnp.int32)
                col = plsc.load_gather(rows2, [ridx, dsp])
                plsc.store_scatter(rows2, [ridx, dsp], col * w16)

    def scatter(b):
        pltpu.sync_copy(rows_v.at[b], acc.at[dst_v.at[b]], add=True)

    edge_issue(ch0, 0)
    edge_issue(ch0 + 1, 1)
    edge_wait(0)
    gather_issue(0)

    @pl.loop(0, NPIPE // 2)
    def _(k):
        for b in (0, 1):
            j = k * 2 + b

            @pl.when(j + 1 < NPIPE)
            def _():
                edge_wait(1 - b)
                gather_issue(1 - b)

            gather_wait(b)
            mul(b)
            scatter(b)

            @pl.when(j + 2 < NPIPE)
            def _():
                edge_issue(ch0 + j + 2, b)

    def tail_chunk(ch, b):
        pltpu.sync_copy(ei_hbm.at[0, ch], src_v.at[b])
        pltpu.sync_copy(ei_hbm.at[1, ch], dst_v.at[b])
        pltpu.sync_copy(w_hbm.at[pl.ds(ch * C, C)], w_v.at[b])
        gather_issue(b)
        gather_wait(b)
        mul(b)
        scatter(b)

    tail_chunk(ch0 + NPIPE, 0)

    @pl.when(wid < CH_REM)
    def _():
        tail_chunk(ch0 + NPIPE + 1, 1)

    plsc.subcore_barrier()

    for j in range(ZC):
        r0 = s * RPW + j * C
        pltpu.sync_copy(acc.at[pl.ds(r0, C)], y_hbm.at[c, pl.ds(r0, C)])
    r0t = s * RPW + ZC * C
    pltpu.sync_copy(acc.at[pl.ds(r0t, ZT)], y_hbm.at[c, pl.ds(r0t, ZT)])


def _combine(y):
    rows = N_NODES * D // 128
    y_r = y.reshape(NC, rows, 128)

    def body(y_ref, o_ref):
        o_ref[...] = y_ref[0] + y_ref[1]

    out = pl.pallas_call(
        body,
        out_shape=jax.ShapeDtypeStruct((rows, 128), jnp.float32),
    )(y_r)
    return out.reshape(N_NODES, D)


PW = BATCH // NW
PG = PW // G


@functools.partial(
    pl.kernel,
    mesh=_mesh,
    out_type=jax.ShapeDtypeStruct((BATCH,), jnp.float32),
    scratch_types=[
        pltpu.VMEM((PW,), jnp.int32),
        pltpu.VMEM((PW,), jnp.int32),
        pltpu.VMEM((PW,), jnp.int32),
        pltpu.VMEM((PW,), jnp.int32),
        pltpu.VMEM((G, D), jnp.float32),
        pltpu.VMEM((G, D), jnp.float32),
        pltpu.VMEM((G, D), jnp.float32),
        pltpu.VMEM((G, D), jnp.float32),
        pltpu.VMEM((PW,), jnp.float32),
        pltpu.SemaphoreType.DMA,
    ],
    compiler_params=_sc_params,
)
def _score(yf_hbm, ui_hbm, ii_hbm, out_hbm, u0v, u1v, i0v, i1v,
           u0r, u1r, i0r, i1r, out_v, sem):
    c = lax.axis_index("c")
    s = lax.axis_index("s")
    wid = c * NS + s
    base = wid * PW

    pltpu.sync_copy(ui_hbm.at[pl.ds(base, PW)], u0v)
    pltpu.sync_copy(ii_hbm.at[pl.ds(base, PW)], i0v)

    @pl.loop(0, PW // D)
    def _(j):
        o = j * D
        u = u0v[pl.ds(o, D)]
        it = i0v[pl.ds(o, D)]
        u1v[pl.ds(o, D)] = u + N_NODES
        i0v[pl.ds(o, D)] = it + N_USERS
        i1v[pl.ds(o, D)] = it + (N_NODES + N_USERS)

    for g in range(PG):
        o = g * G
        descs = [
            pltpu.async_copy(yf_hbm.at[iv.at[pl.ds(o, G)]], rr, sem)
            for iv, rr in ((u0v, u0r), (u1v, u1r), (i0v, i0r), (i1v, i1r))
        ]
        for dsc in descs:
            dsc.wait()
        for p16 in range(G // D):
            ridx = lax.iota(jnp.int32, D) + p16 * D
            acc = jnp.zeros((D,), jnp.float32)
            for d in range(D):
                di = jnp.full((D,), d, jnp.int32)
                uc = (plsc.load_gather(u0r, [ridx, di])
                      + plsc.load_gather(u1r, [ridx, di]))
                ic = (plsc.load_gather(i0r, [ridx, di])
                      + plsc.load_gather(i1r, [ridx, di]))
                acc = acc + uc * ic
            out_v[pl.ds(o + p16 * D, D)] = acc

    pltpu.sync_copy(out_v, out_hbm.at[pl.ds(base, PW)])


def kernel(user_indices, item_indices, user_emb, item_emb, edge_index, edge_weight):
    x = jnp.concatenate([user_emb, item_emb], axis=0)
    ei3 = edge_index.reshape(2, NCHUNKS, C)
    for layer in range(3):
        y = _layer(x, ei3, edge_weight)
        if layer < 2:
            x = _combine(y)
    yf = y.reshape(NC * N_NODES, D)
    return _score(yf, user_indices, item_indices)


# speedup vs baseline: 62.1693x; 3.1615x over previous
import functools

import jax
import jax.numpy as jnp
from jax import lax
from jax.experimental import pallas as pl
from jax.experimental.pallas import tpu as pltpu
from jax.experimental.pallas import tpu_sc as plsc

N_USERS = 50000
N_ITEMS = 50000
N_NODES = N_USERS + N_ITEMS
D = 16
E = 3200000
BATCH = 16384

NC = 2
NS = 16
NW = NC * NS
G = 128
CG = 4
C = G * CG
NCHUNKS = E // C
CH_BASE = NCHUNKS // NW
CH_REM = NCHUNKS - CH_BASE * NW
NPIPE = CH_BASE - 1

RPW = N_NODES // NS
ZC = RPW // C
ZT = RPW - ZC * C

_mesh = plsc.VectorSubcoreMesh(
    core_axis_name="c", subcore_axis_name="s", num_cores=NC, num_subcores=NS)
_sc_params = pltpu.CompilerParams(
    needs_layout_passes=False, use_tc_tiling_on_sc=False)


@functools.partial(
    pl.kernel,
    mesh=_mesh,
    out_type=jax.ShapeDtypeStruct((NC, N_NODES, D), jnp.float32),
    scratch_types=[
        pltpu.VMEM_SHARED((N_NODES, D), jnp.float32),
        pltpu.VMEM((2, C), jnp.int32),
        pltpu.VMEM((2, C), jnp.int32),
        pltpu.VMEM((2, C), jnp.float32),
        pltpu.VMEM((2, C, D), jnp.float32),
        pltpu.SemaphoreType.DMA,
        pltpu.SemaphoreType.DMA,
        pltpu.SemaphoreType.DMA,
    ],
    compiler_params=_sc_params,
)
def _layer(x_hbm, ei_hbm, w_hbm, y_hbm, acc, src_v, dst_v, w_v, rows_v,
           sem_e, sem_g0, sem_g1):
    c = lax.axis_index("c")
    s = lax.axis_index("s")
    wid = c * NS + s
    sem_g = (sem_g0, sem_g1)

    @pl.loop(0, C)
    def _(i):
        rows_v[0, i, :] = jnp.zeros((D,), jnp.float32)

    for j in range(ZC):
        pltpu.sync_copy(rows_v.at[0], acc.at[pl.ds(s * RPW + j * C, C)])
    pltpu.sync_copy(rows_v.at[0].at[pl.ds(0, ZT)],
                    acc.at[pl.ds(s * RPW + ZC * C, ZT)])
    plsc.subcore_barrier()

    ch0 = wid * CH_BASE + jnp.minimum(wid, CH_REM)

    def edge_issue(ch, b):
        pltpu.async_copy(ei_hbm.at[0, ch], src_v.at[b], sem_e)
        pltpu.async_copy(ei_hbm.at[1, ch], dst_v.at[b], sem_e)
        pltpu.async_copy(w_hbm.at[pl.ds(ch * C, C)], w_v.at[b], sem_e)

    def edge_wait(b):
        pltpu.make_async_copy(ei_hbm.at[0, 0], src_v.at[b], sem_e).wait()
        pltpu.make_async_copy(ei_hbm.at[1, 0], dst_v.at[b], sem_e).wait()
        pltpu.make_async_copy(w_hbm.at[pl.ds(0, C)], w_v.at[b], sem_e).wait()

    def gather_issue(b):
        pltpu.async_copy(x_hbm.at[src_v.at[b]], rows_v.at[b], sem_g[b])

    def gather_wait(b):
        pltpu.make_async_copy(x_hbm.at[src_v.at[b]], rows_v.at[b],
                              sem_g[b]).wait()

    def mul(b):
        @pl.loop(0, C // D)
        def _(t):
            w16 = w_v[b, pl.ds(t * D, D)]
            for k in range(D):
                rows_v[b, t * D + k, :] = rows_v[b, t * D + k, :] * w16[k]

    def scatter(b):
        pltpu.sync_copy(rows_v.at[b], acc.at[dst_v.at[b]], add=True)

    edge_issue(ch0, 0)
    edge_issue(ch0 + 1, 1)
    edge_wait(0)
    gather_issue(0)

    @pl.loop(0, NPIPE // 2)
    def _(k):
        for b in (0, 1):
            j = k * 2 + b

            @pl.when(j + 1 < NPIPE)
            def _():
                edge_wait(1 - b)
                gather_issue(1 - b)

            gather_wait(b)
            mul(b)
            scatter(b)

            @pl.when(j + 2 < NPIPE)
            def _():
                edge_issue(ch0 + j + 2, b)

    def tail_chunk(ch, b):
        pltpu.sync_copy(ei_hbm.at[0, ch], src_v.at[b])
        pltpu.sync_copy(ei_hbm.at[1, ch], dst_v.at[b])
        pltpu.sync_copy(w_hbm.at[pl.ds(ch * C, C)], w_v.at[b])
        gather_issue(b)
        gather_wait(b)
        mul(b)
        scatter(b)

    tail_chunk(ch0 + NPIPE, 0)

    @pl.when(wid < CH_REM)
    def _():
        tail_chunk(ch0 + NPIPE + 1, 1)

    plsc.subcore_barrier()

    for j in range(ZC):
        r0 = s * RPW + j * C
        pltpu.sync_copy(acc.at[pl.ds(r0, C)], y_hbm.at[c, pl.ds(r0, C)])
    r0t = s * RPW + ZC * C
    pltpu.sync_copy(acc.at[pl.ds(r0t, ZT)], y_hbm.at[c, pl.ds(r0t, ZT)])


def _combine(y):
    rows = N_NODES * D // 128
    y_r = y.reshape(NC, rows, 128)

    def body(y_ref, o_ref):
        o_ref[...] = y_ref[0] + y_ref[1]

    out = pl.pallas_call(
        body,
        out_shape=jax.ShapeDtypeStruct((rows, 128), jnp.float32),
    )(y_r)
    return out.reshape(N_NODES, D)


PW = BATCH // NW
PG = PW // G


@functools.partial(
    pl.kernel,
    mesh=_mesh,
    out_type=jax.ShapeDtypeStruct((BATCH,), jnp.float32),
    scratch_types=[
        pltpu.VMEM((PW,), jnp.int32),
        pltpu.VMEM((PW,), jnp.int32),
        pltpu.VMEM((PW,), jnp.int32),
        pltpu.VMEM((PW,), jnp.int32),
        pltpu.VMEM((G, D), jnp.float32),
        pltpu.VMEM((G, D), jnp.float32),
        pltpu.VMEM((G, D), jnp.float32),
        pltpu.VMEM((G, D), jnp.float32),
        pltpu.VMEM((PW,), jnp.float32),
        pltpu.SemaphoreType.DMA,
    ],
    compiler_params=_sc_params,
)
def _score(yf_hbm, ui_hbm, ii_hbm, out_hbm, u0v, u1v, i0v, i1v,
           u0r, u1r, i0r, i1r, out_v, sem):
    c = lax.axis_index("c")
    s = lax.axis_index("s")
    wid = c * NS + s
    base = wid * PW

    pltpu.sync_copy(ui_hbm.at[pl.ds(base, PW)], u0v)
    pltpu.sync_copy(ii_hbm.at[pl.ds(base, PW)], i0v)

    @pl.loop(0, PW // D)
    def _(j):
        o = j * D
        u = u0v[pl.ds(o, D)]
        it = i0v[pl.ds(o, D)]
        u1v[pl.ds(o, D)] = u + N_NODES
        i0v[pl.ds(o, D)] = it + N_USERS
        i1v[pl.ds(o, D)] = it + (N_NODES + N_USERS)

    for g in range(PG):
        o = g * G
        descs = [
            pltpu.async_copy(yf_hbm.at[iv.at[pl.ds(o, G)]], rr, sem)
            for iv, rr in ((u0v, u0r), (u1v, u1r), (i0v, i0r), (i1v, i1r))
        ]
        for dsc in descs:
            dsc.wait()
        for p16 in range(G // D):
            ridx = lax.iota(jnp.int32, D) + p16 * D
            acc = jnp.zeros((D,), jnp.float32)
            for d in range(D):
                di = jnp.full((D,), d, jnp.int32)
                uc = (plsc.load_gather(u0r, [ridx, di])
                      + plsc.load_gather(u1r, [ridx, di]))
                ic = (plsc.load_gather(i0r, [ridx, di])
                      + plsc.load_gather(i1r, [ridx, di]))
                acc = acc + uc * ic
            out_v[pl.ds(o + p16 * D, D)] = acc

    pltpu.sync_copy(out_v, out_hbm.at[pl.ds(base, PW)])


def kernel(user_indices, item_indices, user_emb, item_emb, edge_index, edge_weight):
    x = jnp.concatenate([user_emb, item_emb], axis=0)
    ei3 = edge_index.reshape(2, NCHUNKS, C)
    for layer in range(3):
        y = _layer(x, ei3, edge_weight)
        if layer < 2:
            x = _combine(y)
    yf = y.reshape(NC * N_NODES, D)
    return _score(yf, user_indices, item_indices)


# speedup vs baseline: 67.2545x; 1.0818x over previous
import functools

import jax
import jax.numpy as jnp
from jax import lax
from jax.experimental import pallas as pl
from jax.experimental.pallas import tpu as pltpu
from jax.experimental.pallas import tpu_sc as plsc

N_USERS = 50000
N_ITEMS = 50000
N_NODES = N_USERS + N_ITEMS
D = 16
E = 3200000
BATCH = 16384

NC = 2
NS = 16
NW = NC * NS
G = 128
CG = 4
C = G * CG
NCHUNKS = E // C
CH_BASE = NCHUNKS // NW
CH_REM = NCHUNKS - CH_BASE * NW
NPIPE = 192

RPW = N_NODES // NS
ZC = RPW // C
ZT = RPW - ZC * C

_mesh = plsc.VectorSubcoreMesh(
    core_axis_name="c", subcore_axis_name="s", num_cores=NC, num_subcores=NS)
_sc_params = pltpu.CompilerParams(
    needs_layout_passes=False, use_tc_tiling_on_sc=False)


@functools.partial(
    pl.kernel,
    mesh=_mesh,
    out_type=jax.ShapeDtypeStruct((NC, N_NODES, D), jnp.float32),
    scratch_types=[
        pltpu.VMEM_SHARED((N_NODES, D), jnp.float32),
        pltpu.VMEM((2, C), jnp.int32),
        pltpu.VMEM((4, C), jnp.int32),
        pltpu.VMEM((2, C), jnp.float32),
        pltpu.VMEM((3, C, D), jnp.float32),
        pltpu.SemaphoreType.DMA,
        pltpu.SemaphoreType.DMA,
        pltpu.SemaphoreType.DMA,
    ],
    compiler_params=_sc_params,
)
def _layer(x_hbm, ei_hbm, w_hbm, y_hbm, acc, src_v, dst_v, w_v, rows_v,
           sem_e, sem_g, sem_s):
    c = lax.axis_index("c")
    s = lax.axis_index("s")
    wid = c * NS + s

    @pl.loop(0, C)
    def _(i):
        rows_v[0, i, :] = jnp.zeros((D,), jnp.float32)

    for j in range(ZC):
        pltpu.sync_copy(rows_v.at[0], acc.at[pl.ds(s * RPW + j * C, C)])
    pltpu.sync_copy(rows_v.at[0].at[pl.ds(0, ZT)],
                    acc.at[pl.ds(s * RPW + ZC * C, ZT)])
    plsc.subcore_barrier()

    ch0 = wid * CH_BASE + jnp.minimum(wid, CH_REM)

    def edge_issue(ch, p2, p4):
        pltpu.async_copy(ei_hbm.at[0, ch], src_v.at[p2], sem_e)
        pltpu.async_copy(ei_hbm.at[1, ch], dst_v.at[p4], sem_e)
        pltpu.async_copy(w_hbm.at[pl.ds(ch * C, C)], w_v.at[p2], sem_e)

    def edge_wait():
        pltpu.make_async_copy(ei_hbm.at[0, 0], src_v.at[0], sem_e).wait()
        pltpu.make_async_copy(ei_hbm.at[1, 0], dst_v.at[0], sem_e).wait()
        pltpu.make_async_copy(w_hbm.at[pl.ds(0, C)], w_v.at[0], sem_e).wait()

    def gather_issue(p2, p3):
        pltpu.async_copy(x_hbm.at[src_v.at[p2]], rows_v.at[p3], sem_g)

    def gather_wait(p2, p3):
        pltpu.make_async_copy(x_hbm.at[src_v.at[p2]], rows_v.at[p3],
                              sem_g).wait()

    def mul(p3, p2):
        @pl.loop(0, C // D, unroll=2)
        def _(t):
            w16 = w_v[p2, pl.ds(t * D, D)]
            for k in range(D):
                rows_v[p3, t * D + k, :] = rows_v[p3, t * D + k, :] * w16[k]

    def scatter_issue(p3, p4):
        pltpu.async_copy(rows_v.at[p3], acc.at[dst_v.at[p4]], sem_s, add=True)

    def scatter_wait():
        pltpu.make_async_copy(rows_v.at[0], acc.at[dst_v.at[0]],
                              sem_s).wait()

    edge_issue(ch0, 0, 0)
    edge_issue(ch0 + 1, 1, 1)
    edge_wait()
    gather_issue(0, 0)

    @pl.loop(0, NPIPE // 12)
    def _(kk):
        for b in range(12):
            j = kk * 12 + b
            p2, p3, p4 = b % 2, b % 3, b % 4

            gather_wait(p2, p3)

            @pl.when(j + 1 < NPIPE)
            def _():
                edge_wait()
                gather_issue((b + 1) % 2, (b + 1) % 3)

            mul(p3, p2)
            scatter_issue(p3, p4)

            @pl.when(j >= 1)
            def _():
                scatter_wait()

            @pl.when(j + 2 < NPIPE)
            def _():
                edge_issue(ch0 + j + 2, (b + 2) % 2, (b + 2) % 4)

    scatter_wait()

    def tail_chunk(ch):
        pltpu.sync_copy(ei_hbm.at[0, ch], src_v.at[0])
        pltpu.sync_copy(ei_hbm.at[1, ch], dst_v.at[0])
        pltpu.sync_copy(w_hbm.at[pl.ds(ch * C, C)], w_v.at[0])
        gather_issue(0, 0)
        gather_wait(0, 0)
        mul(0, 0)
        pltpu.sync_copy(rows_v.at[0], acc.at[dst_v.at[0]], add=True)

    for t in range(CH_BASE - NPIPE):
        tail_chunk(ch0 + NPIPE + t)

    @pl.when(wid < CH_REM)
    def _():
        tail_chunk(ch0 + CH_BASE)

    plsc.subcore_barrier()

    for j in range(ZC):
        r0 = s * RPW + j * C
        pltpu.sync_copy(acc.at[pl.ds(r0, C)], y_hbm.at[c, pl.ds(r0, C)])
    r0t = s * RPW + ZC * C
    pltpu.sync_copy(acc.at[pl.ds(r0t, ZT)], y_hbm.at[c, pl.ds(r0t, ZT)])


def _combine(y):
    rows = N_NODES * D // 128
    y_r = y.reshape(NC, rows, 128)

    def body(y_ref, o_ref):
        o_ref[...] = y_ref[0] + y_ref[1]

    out = pl.pallas_call(
        body,
        out_shape=jax.ShapeDtypeStruct((rows, 128), jnp.float32),
    )(y_r)
    return out.reshape(N_NODES, D)


PW = BATCH // NW
PG = PW // G


@functools.partial(
    pl.kernel,
    mesh=_mesh,
    out_type=jax.ShapeDtypeStruct((BATCH,), jnp.float32),
    scratch_types=[
        pltpu.VMEM((PW,), jnp.int32),
        pltpu.VMEM((PW,), jnp.int32),
        pltpu.VMEM((PW,), jnp.int32),
        pltpu.VMEM((PW,), jnp.int32),
        pltpu.VMEM((G, D), jnp.float32),
        pltpu.VMEM((G, D), jnp.float32),
        pltpu.VMEM((G, D), jnp.float32),
        pltpu.VMEM((G, D), jnp.float32),
        pltpu.VMEM((PW,), jnp.float32),
        pltpu.SemaphoreType.DMA,
    ],
    compiler_params=_sc_params,
)
def _score(yf_hbm, ui_hbm, ii_hbm, out_hbm, u0v, u1v, i0v, i1v,
           u0r, u1r, i0r, i1r, out_v, sem):
    c = lax.axis_index("c")
    s = lax.axis_index("s")
    wid = c * NS + s
    base = wid * PW

    pltpu.sync_copy(ui_hbm.at[pl.ds(base, PW)], u0v)
    pltpu.sync_copy(ii_hbm.at[pl.ds(base, PW)], i0v)

    @pl.loop(0, PW // D)
    def _(j):
        o = j * D
        u = u0v[pl.ds(o, D)]
        it = i0v[pl.ds(o, D)]
        u1v[pl.ds(o, D)] = u + N_NODES
        i0v[pl.ds(o, D)] = it + N_USERS
        i1v[pl.ds(o, D)] = it + (N_NODES + N_USERS)

    for g in range(PG):
        o = g * G
        descs = [
            pltpu.async_copy(yf_hbm.at[iv.at[pl.ds(o, G)]], rr, sem)
            for iv, rr in ((u0v, u0r), (u1v, u1r), (i0v, i0r), (i1v, i1r))
        ]
        for dsc in descs:
            dsc.wait()
        for p16 in range(G // D):
            ridx = lax.iota(jnp.int32, D) + p16 * D
            acc = jnp.zeros((D,), jnp.float32)
            for d in range(D):
                di = jnp.full((D,), d, jnp.int32)
                uc = (plsc.load_gather(u0r, [ridx, di])
                      + plsc.load_gather(u1r, [ridx, di]))
                ic = (plsc.load_gather(i0r, [ridx, di])
                      + plsc.load_gather(i1r, [ridx, di]))
                acc = acc + uc * ic
            out_v[pl.ds(o + p16 * D, D)] = acc

    pltpu.sync_copy(out_v, out_hbm.at[pl.ds(base, PW)])


def kernel(user_indices, item_indices, user_emb, item_emb, edge_index, edge_weight):
    x = jnp.concatenate([user_emb, item_emb], axis=0)
    ei3 = edge_index.reshape(2, NCHUNKS, C)
    for layer in range(3):
        y = _layer(x, ei3, edge_weight)
        if layer < 2:
            x = _combine(y)
    yf = y.reshape(NC * N_NODES, D)
    return _score(yf, user_indices, item_indices)
